# Initial kernel scaffold; baseline (speedup 1.0000x reference)
import jax
import jax.numpy as jnp
from jax.experimental import pallas as pl


def kernel(adj_matrix, c1_Wx, c1_b, c1_wc, c1_chebW, c1_chebb, c2_Wx, c2_b, c2_wc, c2_chebW, c2_chebb, fc1_W, fc1_b):
    raise NotImplementedError("write your pallas kernel here")


# speedup vs baseline: 1.2175x; 1.2175x over previous
import jax
import jax.numpy as jnp
from jax.experimental import pallas as pl
from jax.experimental.pallas import tpu as pltpu

N = 35
F1 = 140
F2 = 280


def _gclstm_fused_kernel(adj_ref, w1_ref, b1_ref, wc1_ref, cb1_ref,
                         w2_ref, b2_ref, wc2_ref, cb2_ref,
                         fcw_ref, fcb_ref, out_ref):
    X = adj_ref[...]

    def layer(X, w_ref, b_ref, wc_ref, cb_ref):
        gi = jnp.dot(X, w_ref[0], preferred_element_type=jnp.float32)
        gt = jnp.dot(X, w_ref[2], preferred_element_type=jnp.float32)
        go = jnp.dot(X, w_ref[3], preferred_element_type=jnp.float32)
        I = jax.nn.sigmoid(gi + b_ref[0] + cb_ref[0])
        T = jnp.tanh(gt + b_ref[2] + cb_ref[2])
        C = I * T
        O = jax.nn.sigmoid(go + b_ref[3] + cb_ref[3] + wc_ref[2] * C)
        return jax.nn.relu(O * jnp.tanh(C))

    H1 = layer(X, w1_ref, b1_ref, wc1_ref, cb1_ref)
    H2 = layer(H1, w2_ref, b2_ref, wc2_ref, cb2_ref)
    Y = jnp.dot(H2, fcw_ref[...], preferred_element_type=jnp.float32)
    out_ref[...] = jax.nn.relu(Y + fcb_ref[...])


def kernel(adj_matrix, c1_Wx, c1_b, c1_wc, c1_chebW, c1_chebb,
           c2_Wx, c2_b, c2_wc, c2_chebW, c2_chebb, fc1_W, fc1_b):
    del c1_chebW, c2_chebW
    b1 = c1_b[:, 0, :]
    b2 = c2_b[:, 0, :]
    wc1 = c1_wc[:, 0, :]
    wc2 = c2_wc[:, 0, :]
    fcb = fc1_b[None, :]
    return pl.pallas_call(
        _gclstm_fused_kernel,
        out_shape=jax.ShapeDtypeStruct((N, N), jnp.float32),
    )(adj_matrix, c1_Wx, b1, wc1, c1_chebb,
      c2_Wx, b2, wc2, c2_chebb, fc1_W, fcb)


# speedup vs baseline: 2.1433x; 1.7604x over previous
import jax
import jax.numpy as jnp
from jax.experimental import pallas as pl
from jax.experimental.pallas import tpu as pltpu

N = 35
F1 = 140
F2 = 280


def _gclstm_fused_kernel(adj_ref, w1_ref, b1_ref, wc1_ref, cb1_ref,
                         w2_ref, b2_ref, wc2_ref, cb2_ref,
                         fcw_ref, fcb_ref, out_ref):
    X = adj_ref[...]

    def layer(X, w_ref, b_ref, wc_ref, cb_ref):
        gi = jnp.dot(X, w_ref[0], preferred_element_type=jnp.float32)
        gt = jnp.dot(X, w_ref[2], preferred_element_type=jnp.float32)
        go = jnp.dot(X, w_ref[3], preferred_element_type=jnp.float32)
        I = jax.nn.sigmoid(gi + (b_ref[0] + cb_ref[0][None, :]))
        T = jnp.tanh(gt + (b_ref[2] + cb_ref[2][None, :]))
        C = I * T
        O = jax.nn.sigmoid(go + (b_ref[3] + cb_ref[3][None, :]) + wc_ref[2] * C)
        return jax.nn.relu(O * jnp.tanh(C))

    H1 = layer(X, w1_ref, b1_ref, wc1_ref, cb1_ref)
    H2 = layer(H1, w2_ref, b2_ref, wc2_ref, cb2_ref)
    Y = jnp.dot(H2, fcw_ref[...], preferred_element_type=jnp.float32)
    out_ref[...] = jax.nn.relu(Y + fcb_ref[...][None, :])


def kernel(adj_matrix, c1_Wx, c1_b, c1_wc, c1_chebW, c1_chebb,
           c2_Wx, c2_b, c2_wc, c2_chebW, c2_chebb, fc1_W, fc1_b):
    del c1_chebW, c2_chebW
    return pl.pallas_call(
        _gclstm_fused_kernel,
        out_shape=jax.ShapeDtypeStruct((N, N), jnp.float32),
    )(adj_matrix, c1_Wx, c1_b, c1_wc, c1_chebb,
      c2_Wx, c2_b, c2_wc, c2_chebb, fc1_W, fc1_b)
